# baseline (speedup 1.0000x reference)
import functools

import jax
import jax.numpy as jnp
from jax import lax
from jax.experimental import pallas as pl
from jax.experimental.pallas import tpu as pltpu
from jax.experimental.pallas import tpu_sc as plsc

_NC = 2
_NS = 16
_NW = _NC * _NS
_BG = 4
_SG = 8
_IDX_CHUNK = 128
_VBLK = 8192


def _tc_pack_table(table_t):
    D, V = table_t.shape
    g = 128 // D
    v_blk = _VBLK
    rows_blk = v_blk // g
    grid = (pl.cdiv(V, v_blk),)
    n_rows = grid[0] * rows_blk

    def pack_kernel(in_ref, out_ref):
        a = in_ref[...]
        for q in range(g):
            out_ref[:, q * D:(q + 1) * D] = a[:, q * rows_blk:(q + 1) * rows_blk].T

    return pl.pallas_call(
        pack_kernel,
        grid=grid,
        in_specs=[pl.BlockSpec((D, v_blk), lambda i: (0, i))],
        out_specs=pl.BlockSpec((rows_blk, 128), lambda i: (i, 0)),
        out_shape=jax.ShapeDtypeStruct((n_rows, 128), table_t.dtype),
    )(table_t)


def _sc_gather(features, table_lin):
    S, B = features.shape
    V, D = table_lin.shape
    s_per = S // _SG
    b_per = B // _BG
    n_ch = b_per // _IDX_CHUNK
    n_i = D // 8
    n_j = b_per // 128

    mesh = plsc.VectorSubcoreMesh(core_axis_name="c", subcore_axis_name="s")

    @functools.partial(
        pl.kernel,
        out_type=jax.ShapeDtypeStruct((S, n_i, B // 128, 8, 128), jnp.float32),
        mesh=mesh,
        scratch_types=[
            pltpu.VMEM((s_per, b_per), jnp.int32),
            pltpu.VMEM((b_per, D), jnp.float32),
            pltpu.VMEM((2, n_i, n_j, 8, 128), jnp.float32),
            pltpu.SemaphoreType.DMA,
            pltpu.SemaphoreType.DMA,
        ],
        compiler_params=pltpu.CompilerParams(
            use_tc_tiling_on_sc=False, needs_layout_passes=False
        ),
    )
    def gather_kernel(feat_hbm, table_hbm, out_hbm, idx_v, rows_v, tile_v,
                      gsem, wsem):
        wid = lax.axis_index("c") * _NS + lax.axis_index("s")
        bg = wid // _SG
        sg = wid % _SG
        s0 = sg * s_per
        b0 = bg * b_per
        iota16 = lax.broadcasted_iota(jnp.int32, (16,), 0)

        pltpu.sync_copy(
            feat_hbm.at[pl.ds(s0, s_per), pl.ds(b0, b_per)], idx_v
        )

        n_vec = b_per // 16
        def remap(t, carry):
            s_l = t // n_vec
            i = t % n_vec
            v = idx_v[s_l, pl.ds(i * 16, 16)]
            k = (v & -8192) | ((v & 2047) << 2) | ((v >> 11) & 3)
            idx_v[s_l, pl.ds(i * 16, 16)] = k
            return carry

        lax.fori_loop(0, s_per * n_vec, remap, None)

        def write_descs(s_l, buf):
            return [
                pltpu.make_async_copy(
                    tile_v.at[buf, i],
                    out_hbm.at[s0 + s_l, i, pl.ds(n_j * bg, n_j)],
                    wsem,
                )
                for i in range(n_i)
            ]

        def step(s_l, carry):
            buf = lax.rem(s_l, 2)
            descs = [
                pltpu.async_copy(
                    table_hbm.at[idx_v.at[s_l, pl.ds(j * _IDX_CHUNK, _IDX_CHUNK)]],
                    rows_v.at[pl.ds(j * _IDX_CHUNK, _IDX_CHUNK)],
                    gsem,
                )
                for j in range(n_ch)
            ]
            for d in descs:
                d.wait()

            @pl.when(s_l >= 2)
            def _():
                for d in write_descs(s_l - 2, buf):
                    d.wait()

            def tr(t, carry2):
                i = t >> 6
                jl = (t >> 3) & 7
                q = t & 7
                rbase = 128 * jl
                cvec = jnp.broadcast_to(8 * i + q, (16,))
                for ci in range(8):
                    rvec = rbase + ci * 16 + iota16
                    vals = plsc.load_gather(rows_v, [rvec, cvec])
                    tile_v[buf, i, jl, q, pl.ds(ci * 16, 16)] = vals
                return carry2

            lax.fori_loop(0, n_i * n_j * 8, tr, None)

            for d in write_descs(s_l, buf):
                d.start()
            return carry

        lax.fori_loop(0, s_per, step, None)
        for d in write_descs(s_per - 2, lax.rem(s_per - 2, 2)):
            d.wait()
        for d in write_descs(s_per - 1, lax.rem(s_per - 1, 2)):
            d.wait()

    return gather_kernel(features, table_lin)


def _tc_tile_init(init_state, batch):
    L, _, H = init_state.shape
    blk = 512
    grid = (batch // blk,)

    def tile_kernel(init_ref, out_ref):
        out_ref[...] = jnp.broadcast_to(init_ref[...], out_ref.shape)

    return pl.pallas_call(
        tile_kernel,
        grid=grid,
        in_specs=[pl.BlockSpec((L, 1, H), lambda i: (0, 0, 0))],
        out_specs=pl.BlockSpec((L, blk, H), lambda i: (0, i, 0)),
        out_shape=jax.ShapeDtypeStruct((L, batch, H), init_state.dtype),
    )(init_state)


def kernel(features, lengths, table, init_state):
    del lengths
    V, D = table.shape
    t_pack = _tc_pack_table(jnp.transpose(table))
    table_lin = jnp.reshape(t_pack, (t_pack.shape[0] * (128 // D), D))
    B = features.shape[1]
    emb5 = _sc_gather(features, table_lin)
    emb = jnp.transpose(emb5, (2, 4, 0, 1, 3)).reshape(B, features.shape[0], D)
    init = _tc_tile_init(init_state, features.shape[1])
    return (emb, init)


# speedup vs baseline: 2.2814x; 2.2814x over previous
import functools

import jax
import jax.numpy as jnp
from jax import lax
from jax.experimental import pallas as pl
from jax.experimental.pallas import tpu as pltpu
from jax.experimental.pallas import tpu_sc as plsc

_NC = 2
_NS = 16
_NW = _NC * _NS
_BG = 4
_SG = 8
_IDX_CHUNK = 128
_VBLK = 8192


def _tc_pack_table(table_t):
    D, V = table_t.shape
    g = 128 // D
    v_blk = _VBLK
    rows_blk = v_blk // g
    grid = (pl.cdiv(V, v_blk),)
    n_rows = grid[0] * rows_blk

    def pack_kernel(in_ref, out_ref):
        a = in_ref[...]
        for q in range(g):
            out_ref[:, q * D:(q + 1) * D] = a[:, q * rows_blk:(q + 1) * rows_blk].T

    return pl.pallas_call(
        pack_kernel,
        grid=grid,
        in_specs=[pl.BlockSpec((D, v_blk), lambda i: (0, i))],
        out_specs=pl.BlockSpec((rows_blk, 128), lambda i: (i, 0)),
        out_shape=jax.ShapeDtypeStruct((n_rows, 128), table_t.dtype),
    )(table_t)


def _sc_gather(features, table_lin):
    S, B = features.shape
    V, D = table_lin.shape
    s_per = S // _SG
    b_per = B // _BG
    n_ch = b_per // _IDX_CHUNK
    n_i = D // 8
    n_j = b_per // 128
    TP = 129

    mesh = plsc.VectorSubcoreMesh(core_axis_name="c", subcore_axis_name="s")

    @functools.partial(
        pl.kernel,
        out_type=jax.ShapeDtypeStruct((S, n_i, (B // 128) * 8, 128), jnp.float32),
        mesh=mesh,
        scratch_types=[
            pltpu.VMEM((2, b_per), jnp.int32),
            pltpu.VMEM((2, b_per, D), jnp.float32),
            pltpu.VMEM((n_i, n_j * 8, TP), jnp.float32),
            pltpu.SemaphoreType.DMA,
            pltpu.SemaphoreType.DMA,
            pltpu.SemaphoreType.DMA,
        ],
        compiler_params=pltpu.CompilerParams(
            use_tc_tiling_on_sc=False, needs_layout_passes=False
        ),
    )
    def gather_kernel(feat_hbm, table_hbm, out_hbm, idx_v, rows_v, tile_v,
                      isem, gsem, wsem):
        wid = lax.axis_index("c") * _NS + lax.axis_index("s")
        bg = wid // _SG
        sg = wid % _SG
        s0 = sg * s_per
        b0 = bg * b_per
        iota16 = lax.broadcasted_iota(jnp.int32, (16,), 0)
        I_LO = iota16 >> 3
        I_HI = I_LO + 2
        QV = iota16 & 7

        def idx_desc(s_l):
            return pltpu.make_async_copy(
                feat_hbm.at[s0 + s_l, pl.ds(b0, b_per)],
                idx_v.at[lax.rem(s_l, 2)],
                isem,
            )

        def remap(s_l):
            slot = lax.rem(s_l, 2)

            def body(i, carry):
                v = idx_v[slot, pl.ds(i * 16, 16)]
                k = (v & -8192) | ((v & 2047) << 2) | ((v >> 11) & 3)
                idx_v[slot, pl.ds(i * 16, 16)] = k
                return carry

            lax.fori_loop(0, b_per // 16, body, None)

        def gather_descs(s_l):
            slot = lax.rem(s_l, 2)
            return [
                pltpu.make_async_copy(
                    table_hbm.at[idx_v.at[slot, pl.ds(j * _IDX_CHUNK, _IDX_CHUNK)]],
                    rows_v.at[slot, pl.ds(j * _IDX_CHUNK, _IDX_CHUNK)],
                    gsem,
                )
                for j in range(n_ch)
            ]

        def write_descs(s_l):
            return [
                pltpu.make_async_copy(
                    tile_v.at[i, :, pl.ds(0, 128)],
                    out_hbm.at[s0 + s_l, i, pl.ds(n_j * 8 * bg, n_j * 8)],
                    wsem,
                )
                for i in range(n_i)
            ]

        idx_desc(0).start()
        idx_desc(0).wait()
        remap(0)
        idx_desc(1).start()
        for d in gather_descs(0):
            d.start()

        def step(s_l, carry):
            rbuf = lax.rem(s_l, 2)

            @pl.when(s_l < s_per - 1)
            def _():
                idx_desc(s_l + 1).wait()
                remap(s_l + 1)

            for d in gather_descs(s_l):
                d.wait()

            @pl.when(s_l < s_per - 1)
            def _():
                for d in gather_descs(s_l + 1):
                    d.start()

            @pl.when(s_l < s_per - 2)
            def _():
                idx_desc(s_l + 2).start()

            @pl.when(s_l >= 1)
            def _():
                for d in write_descs(s_l - 1):
                    d.wait()

            def tr(t, carry2):
                for bi in range(4):
                    b = t * 4 + bi
                    jl = b >> 7
                    c = b & 127
                    cvec = jnp.broadcast_to(c, (16,))
                    midv = jnp.broadcast_to(jl * 8, (16,)) + QV
                    va = rows_v[rbuf, b, pl.ds(0, 16)]
                    vb = rows_v[rbuf, b, pl.ds(16, 16)]
                    plsc.store_scatter(tile_v, [I_LO, midv, cvec], va)
                    plsc.store_scatter(tile_v, [I_HI, midv, cvec], vb)
                return carry2

            lax.fori_loop(0, b_per // 4, tr, None)

            for d in write_descs(s_l):
                d.start()
            return carry

        lax.fori_loop(0, s_per, step, None)
        for d in write_descs(s_per - 1):
            d.wait()

    return gather_kernel(features, table_lin)


def _tc_tile_init(init_state, batch):
    L, _, H = init_state.shape
    blk = 512
    grid = (batch // blk,)

    def tile_kernel(init_ref, out_ref):
        out_ref[...] = jnp.broadcast_to(init_ref[...], out_ref.shape)

    return pl.pallas_call(
        tile_kernel,
        grid=grid,
        in_specs=[pl.BlockSpec((L, 1, H), lambda i: (0, 0, 0))],
        out_specs=pl.BlockSpec((L, blk, H), lambda i: (0, i, 0)),
        out_shape=jax.ShapeDtypeStruct((L, batch, H), init_state.dtype),
    )(init_state)


def kernel(features, lengths, table, init_state):
    del lengths
    V, D = table.shape
    t_pack = _tc_pack_table(jnp.transpose(table))
    table_lin = jnp.reshape(t_pack, (t_pack.shape[0] * (128 // D), D))
    B = features.shape[1]
    S = features.shape[0]
    out4 = _sc_gather(features, table_lin)
    emb5 = jnp.reshape(out4, (S, D // 8, B // 128, 8, 128))
    emb = jnp.transpose(emb5, (2, 4, 0, 1, 3)).reshape(B, S, D)
    init = _tc_tile_init(init_state, features.shape[1])
    return (emb, init)


# speedup vs baseline: 2.2837x; 1.0010x over previous
import functools

import jax
import jax.numpy as jnp
from jax import lax
from jax.experimental import pallas as pl
from jax.experimental.pallas import tpu as pltpu
from jax.experimental.pallas import tpu_sc as plsc

_NC = 2
_NS = 16
_NW = _NC * _NS
_BG = 4
_SG = 8
_IDX_CHUNK = 512
_VBLK = 8192
_VQ = _VBLK // 4
_VQ_LOG2 = _VQ.bit_length() - 1


def _tc_pack_table(table_t):
    D, V = table_t.shape
    g = 128 // D
    v_blk = _VBLK
    rows_blk = v_blk // g
    grid = (pl.cdiv(V, v_blk),)
    n_rows = grid[0] * rows_blk

    def pack_kernel(in_ref, out_ref):
        a = in_ref[...]
        for q in range(g):
            out_ref[:, q * D:(q + 1) * D] = a[:, q * rows_blk:(q + 1) * rows_blk].T

    return pl.pallas_call(
        pack_kernel,
        grid=grid,
        in_specs=[pl.BlockSpec((D, v_blk), lambda i: (0, i))],
        out_specs=pl.BlockSpec((rows_blk, 128), lambda i: (i, 0)),
        out_shape=jax.ShapeDtypeStruct((n_rows, 128), table_t.dtype),
    )(table_t)


def _sc_gather(features, table_lin):
    S, B = features.shape
    V, D = table_lin.shape
    s_per = S // _SG
    b_per = B // _BG
    n_ch = b_per // _IDX_CHUNK
    n_i = D // 8
    n_j = b_per // 128
    TP = 129

    mesh = plsc.VectorSubcoreMesh(core_axis_name="c", subcore_axis_name="s")

    @functools.partial(
        pl.kernel,
        out_type=jax.ShapeDtypeStruct((S, n_i, (B // 128) * 8, 128), jnp.float32),
        mesh=mesh,
        scratch_types=[
            pltpu.VMEM((2, b_per), jnp.int32),
            pltpu.VMEM((2, b_per, D), jnp.float32),
            pltpu.VMEM((n_i, n_j * 8, TP), jnp.float32),
            pltpu.SemaphoreType.DMA,
            pltpu.SemaphoreType.DMA,
            pltpu.SemaphoreType.DMA,
        ],
        compiler_params=pltpu.CompilerParams(
            use_tc_tiling_on_sc=False, needs_layout_passes=False
        ),
    )
    def gather_kernel(feat_hbm, table_hbm, out_hbm, idx_v, rows_v, tile_v,
                      isem, gsem, wsem):
        wid = lax.axis_index("c") * _NS + lax.axis_index("s")
        bg = wid // _SG
        sg = wid % _SG
        s0 = sg * s_per
        b0 = bg * b_per
        iota16 = lax.broadcasted_iota(jnp.int32, (16,), 0)
        I_LO = iota16 >> 3
        I_HI = I_LO + 2
        QV = iota16 & 7

        def idx_desc(s_l):
            return pltpu.make_async_copy(
                feat_hbm.at[s0 + s_l, pl.ds(b0, b_per)],
                idx_v.at[lax.rem(s_l, 2)],
                isem,
            )

        def remap(s_l):
            slot = lax.rem(s_l, 2)

            def body(i, carry):
                v = idx_v[slot, pl.ds(i * 16, 16)]
                k = (v & -_VBLK) | ((v & (_VQ - 1)) << 2) | ((v >> _VQ_LOG2) & 3)
                idx_v[slot, pl.ds(i * 16, 16)] = k
                return carry

            lax.fori_loop(0, b_per // 16, body, None)

        def gather_descs(s_l):
            slot = lax.rem(s_l, 2)
            return [
                pltpu.make_async_copy(
                    table_hbm.at[idx_v.at[slot, pl.ds(j * _IDX_CHUNK, _IDX_CHUNK)]],
                    rows_v.at[slot, pl.ds(j * _IDX_CHUNK, _IDX_CHUNK)],
                    gsem,
                )
                for j in range(n_ch)
            ]

        def write_descs(s_l):
            return [
                pltpu.make_async_copy(
                    tile_v.at[i, :, pl.ds(0, 128)],
                    out_hbm.at[s0 + s_l, i, pl.ds(n_j * 8 * bg, n_j * 8)],
                    wsem,
                )
                for i in range(n_i)
            ]

        idx_desc(0).start()
        idx_desc(0).wait()
        remap(0)
        idx_desc(1).start()
        for d in gather_descs(0):
            d.start()

        def step(s_l, carry):
            rbuf = lax.rem(s_l, 2)

            @pl.when(s_l < s_per - 1)
            def _():
                idx_desc(s_l + 1).wait()
                remap(s_l + 1)

            for d in gather_descs(s_l):
                d.wait()

            @pl.when(s_l < s_per - 1)
            def _():
                for d in gather_descs(s_l + 1):
                    d.start()

            @pl.when(s_l < s_per - 2)
            def _():
                idx_desc(s_l + 2).start()

            @pl.when(s_l >= 1)
            def _():
                for d in write_descs(s_l - 1):
                    d.wait()

            def tr(t, carry2):
                for bi in range(4):
                    b = t * 4 + bi
                    jl = b >> 7
                    c = b & 127
                    cvec = jnp.broadcast_to(c, (16,))
                    midv = jnp.broadcast_to(jl * 8, (16,)) + QV
                    va = rows_v[rbuf, b, pl.ds(0, 16)]
                    vb = rows_v[rbuf, b, pl.ds(16, 16)]
                    plsc.store_scatter(tile_v, [I_LO, midv, cvec], va)
                    plsc.store_scatter(tile_v, [I_HI, midv, cvec], vb)
                return carry2

            lax.fori_loop(0, b_per // 4, tr, None)

            for d in write_descs(s_l):
                d.start()
            return carry

        lax.fori_loop(0, s_per, step, None)
        for d in write_descs(s_per - 1):
            d.wait()

    return gather_kernel(features, table_lin)


def _tc_tile_init(init_state, batch):
    L, _, H = init_state.shape
    blk = 512
    grid = (batch // blk,)

    def tile_kernel(init_ref, out_ref):
        out_ref[...] = jnp.broadcast_to(init_ref[...], out_ref.shape)

    return pl.pallas_call(
        tile_kernel,
        grid=grid,
        in_specs=[pl.BlockSpec((L, 1, H), lambda i: (0, 0, 0))],
        out_specs=pl.BlockSpec((L, blk, H), lambda i: (0, i, 0)),
        out_shape=jax.ShapeDtypeStruct((L, batch, H), init_state.dtype),
    )(init_state)


def kernel(features, lengths, table, init_state):
    del lengths
    V, D = table.shape
    t_pack = _tc_pack_table(jnp.transpose(table))
    table_lin = jnp.reshape(t_pack, (t_pack.shape[0] * (128 // D), D))
    B = features.shape[1]
    S = features.shape[0]
    out4 = _sc_gather(features, table_lin)
    emb5 = jnp.reshape(out4, (S, D // 8, B // 128, 8, 128))
    emb = jnp.transpose(emb5, (2, 4, 0, 1, 3)).reshape(B, S, D)
    init = _tc_tile_init(init_state, features.shape[1])
    return (emb, init)


# speedup vs baseline: 3.0655x; 1.3423x over previous
import functools

import jax
import jax.numpy as jnp
from jax import lax
from jax.experimental import pallas as pl
from jax.experimental.pallas import tpu as pltpu
from jax.experimental.pallas import tpu_sc as plsc

_NC = 2
_NS = 16
_NW = _NC * _NS
_BG = 4
_SG = 8
_IDX_CHUNK = 512
_PBLK = 8192
_VBLK = 512
_VQ = _VBLK // 4
_VQ_LOG2 = _VQ.bit_length() - 1


def _tc_pack_table(table_t):
    D, V = table_t.shape
    g = 128 // D
    v_blk = _PBLK
    rows_blk = v_blk // g
    grid = (pl.cdiv(V, v_blk),)
    n_rows = grid[0] * rows_blk

    def pack_kernel(in_ref, out_ref):
        a = in_ref[...]
        for w in range(v_blk // (4 * 128)):
            sub = a[:, w * 512:(w + 1) * 512]
            c = jnp.concatenate(
                [sub[:, m * 128:(m + 1) * 128] for m in range(4)], axis=0
            )
            out_ref[w * 128:(w + 1) * 128, :] = c.T

    return pl.pallas_call(
        pack_kernel,
        grid=grid,
        in_specs=[pl.BlockSpec((D, v_blk), lambda i: (0, i))],
        out_specs=pl.BlockSpec((rows_blk, 128), lambda i: (i, 0)),
        out_shape=jax.ShapeDtypeStruct((n_rows, 128), table_t.dtype),
    )(table_t)


def _sc_gather(features, table_lin):
    S, B = features.shape
    V, D = table_lin.shape
    s_per = S // _SG
    b_per = B // _BG
    n_ch = b_per // _IDX_CHUNK
    n_i = D // 8
    n_j = b_per // 128
    TP = 129

    mesh = plsc.VectorSubcoreMesh(core_axis_name="c", subcore_axis_name="s")

    @functools.partial(
        pl.kernel,
        out_type=jax.ShapeDtypeStruct((S, n_i, (B // 128) * 8, 128), jnp.float32),
        mesh=mesh,
        scratch_types=[
            pltpu.VMEM((2, b_per), jnp.int32),
            pltpu.VMEM((2, b_per, D), jnp.float32),
            pltpu.VMEM((n_i, n_j * 8, TP), jnp.float32),
            pltpu.SemaphoreType.DMA,
            pltpu.SemaphoreType.DMA,
            pltpu.SemaphoreType.DMA,
        ],
        compiler_params=pltpu.CompilerParams(
            use_tc_tiling_on_sc=False, needs_layout_passes=False
        ),
    )
    def gather_kernel(feat_hbm, table_hbm, out_hbm, idx_v, rows_v, tile_v,
                      isem, gsem, wsem):
        wid = lax.axis_index("c") * _NS + lax.axis_index("s")
        bg = wid // _SG
        sg = wid % _SG
        s0 = sg * s_per
        b0 = bg * b_per
        iota16 = lax.broadcasted_iota(jnp.int32, (16,), 0)
        I_LO = iota16 >> 3
        I_HI = I_LO + 2
        QV = iota16 & 7

        def idx_desc(s_l):
            return pltpu.make_async_copy(
                feat_hbm.at[s0 + s_l, pl.ds(b0, b_per)],
                idx_v.at[lax.rem(s_l, 2)],
                isem,
            )

        def remap(s_l):
            slot = lax.rem(s_l, 2)

            def body(i, carry):
                v = idx_v[slot, pl.ds(i * 16, 16)]
                k = (v & -_VBLK) | ((v & (_VQ - 1)) << 2) | ((v >> _VQ_LOG2) & 3)
                idx_v[slot, pl.ds(i * 16, 16)] = k
                return carry

            lax.fori_loop(0, b_per // 16, body, None)

        def gather_descs(s_l):
            slot = lax.rem(s_l, 2)
            return [
                pltpu.make_async_copy(
                    table_hbm.at[idx_v.at[slot, pl.ds(j * _IDX_CHUNK, _IDX_CHUNK)]],
                    rows_v.at[slot, pl.ds(j * _IDX_CHUNK, _IDX_CHUNK)],
                    gsem,
                )
                for j in range(n_ch)
            ]

        def write_descs(s_l):
            return [
                pltpu.make_async_copy(
                    tile_v.at[i, :, pl.ds(0, 128)],
                    out_hbm.at[s0 + s_l, i, pl.ds(n_j * 8 * bg, n_j * 8)],
                    wsem,
                )
                for i in range(n_i)
            ]

        idx_desc(0).start()
        idx_desc(0).wait()
        remap(0)
        idx_desc(1).start()
        for d in gather_descs(0):
            d.start()

        def step(s_l, carry):
            rbuf = lax.rem(s_l, 2)

            @pl.when(s_l < s_per - 1)
            def _():
                idx_desc(s_l + 1).wait()
                remap(s_l + 1)

            for d in gather_descs(s_l):
                d.wait()

            @pl.when(s_l < s_per - 1)
            def _():
                for d in gather_descs(s_l + 1):
                    d.start()

            @pl.when(s_l < s_per - 2)
            def _():
                idx_desc(s_l + 2).start()

            @pl.when(s_l >= 1)
            def _():
                for d in write_descs(s_l - 1):
                    d.wait()

            def tr(t, carry2):
                for bi in range(4):
                    b = t * 4 + bi
                    jl = b >> 7
                    c = b & 127
                    cvec = jnp.broadcast_to(c, (16,))
                    midv = jnp.broadcast_to(jl * 8, (16,)) + QV
                    va = rows_v[rbuf, b, pl.ds(0, 16)]
                    vb = rows_v[rbuf, b, pl.ds(16, 16)]
                    plsc.store_scatter(tile_v, [I_LO, midv, cvec], va)
                    plsc.store_scatter(tile_v, [I_HI, midv, cvec], vb)
                return carry2

            lax.fori_loop(0, b_per // 4, tr, None)

            for d in write_descs(s_l):
                d.start()
            return carry

        lax.fori_loop(0, s_per, step, None)
        for d in write_descs(s_per - 1):
            d.wait()

    return gather_kernel(features, table_lin)


def _tc_tile_init(init_state, batch):
    L, _, H = init_state.shape
    blk = 512
    grid = (batch // blk,)

    def tile_kernel(init_ref, out_ref):
        out_ref[...] = jnp.broadcast_to(init_ref[...], out_ref.shape)

    return pl.pallas_call(
        tile_kernel,
        grid=grid,
        in_specs=[pl.BlockSpec((L, 1, H), lambda i: (0, 0, 0))],
        out_specs=pl.BlockSpec((L, blk, H), lambda i: (0, i, 0)),
        out_shape=jax.ShapeDtypeStruct((L, batch, H), init_state.dtype),
    )(init_state)


def kernel(features, lengths, table, init_state):
    del lengths
    V, D = table.shape
    t_pack = _tc_pack_table(jnp.transpose(table))
    table_lin = jnp.reshape(t_pack, (t_pack.shape[0] * (128 // D), D))
    B = features.shape[1]
    S = features.shape[0]
    out4 = _sc_gather(features, table_lin)
    emb5 = jnp.reshape(out4, (S, D // 8, B // 128, 8, 128))
    emb = jnp.transpose(emb5, (2, 4, 0, 1, 3)).reshape(B, S, D)
    init = _tc_tile_init(init_state, features.shape[1])
    return (emb, init)


# speedup vs baseline: 3.0713x; 1.0019x over previous
import functools

import jax
import jax.numpy as jnp
from jax import lax
from jax.experimental import pallas as pl
from jax.experimental.pallas import tpu as pltpu
from jax.experimental.pallas import tpu_sc as plsc

_NC = 2
_NS = 16
_NW = _NC * _NS
_BG = 4
_SG = 8
_IDX_CHUNK = 512
_PBLK = 8192
_VBLK = 512
_VQ = _VBLK // 4
_VQ_LOG2 = _VQ.bit_length() - 1


def _tc_pack_table(table_t):
    D, V = table_t.shape
    g = 128 // D
    v_blk = _PBLK
    rows_blk = v_blk // g
    grid = (pl.cdiv(V, v_blk),)
    n_rows = grid[0] * rows_blk

    def pack_kernel(in_ref, out_ref):
        a = in_ref[...]
        for w in range(v_blk // (4 * 128)):
            sub = a[:, w * 512:(w + 1) * 512]
            c = jnp.concatenate(
                [sub[:, m * 128:(m + 1) * 128] for m in range(4)], axis=0
            )
            out_ref[w * 128:(w + 1) * 128, :] = c.T

    return pl.pallas_call(
        pack_kernel,
        grid=grid,
        in_specs=[pl.BlockSpec((D, v_blk), lambda i: (0, i))],
        out_specs=pl.BlockSpec((rows_blk, 128), lambda i: (i, 0)),
        out_shape=jax.ShapeDtypeStruct((n_rows, 128), table_t.dtype),
    )(table_t)


def _sc_gather(features, table_lin):
    S, B = features.shape
    V, D = table_lin.shape
    s_per = S // _SG
    b_per = B // _BG
    n_ch = b_per // _IDX_CHUNK
    n_i = D // 8
    n_j = b_per // 128
    TP = 129

    mesh = plsc.VectorSubcoreMesh(core_axis_name="c", subcore_axis_name="s")

    @functools.partial(
        pl.kernel,
        out_type=jax.ShapeDtypeStruct((S, n_i, (B // 128) * 8, 128), jnp.float32),
        mesh=mesh,
        scratch_types=[
            pltpu.VMEM((2, b_per), jnp.int32),
            pltpu.VMEM((2, b_per, D), jnp.float32),
            pltpu.VMEM((n_i, n_j * 8, TP), jnp.float32),
            pltpu.SemaphoreType.DMA,
            pltpu.SemaphoreType.DMA((2,)),
            pltpu.SemaphoreType.DMA,
        ],
        compiler_params=pltpu.CompilerParams(
            use_tc_tiling_on_sc=False, needs_layout_passes=False
        ),
    )
    def gather_kernel(feat_hbm, table_hbm, out_hbm, idx_v, rows_v, tile_v,
                      isem, gsem, wsem):
        wid = lax.axis_index("c") * _NS + lax.axis_index("s")
        bg = wid // _SG
        sg = wid % _SG
        s0 = sg * s_per
        b0 = bg * b_per
        iota16 = lax.broadcasted_iota(jnp.int32, (16,), 0)
        I_LO = iota16 >> 3
        I_HI = I_LO + 2
        QV = iota16 & 7

        def idx_desc(s_l):
            return pltpu.make_async_copy(
                feat_hbm.at[s0 + s_l, pl.ds(b0, b_per)],
                idx_v.at[lax.rem(s_l, 2)],
                isem,
            )

        def remap(s_l):
            slot = lax.rem(s_l, 2)

            def body(i, carry):
                v = idx_v[slot, pl.ds(i * 16, 16)]
                k = (v & -_VBLK) | ((v & (_VQ - 1)) << 2) | ((v >> _VQ_LOG2) & 3)
                idx_v[slot, pl.ds(i * 16, 16)] = k
                return carry

            lax.fori_loop(0, b_per // 16, body, None)

        def gather_descs(s_l):
            slot = lax.rem(s_l, 2)
            return [
                pltpu.make_async_copy(
                    table_hbm.at[idx_v.at[slot, pl.ds(j * _IDX_CHUNK, _IDX_CHUNK)]],
                    rows_v.at[slot, pl.ds(j * _IDX_CHUNK, _IDX_CHUNK)],
                    gsem.at[slot],
                )
                for j in range(n_ch)
            ]

        def write_descs(s_l):
            return [
                pltpu.make_async_copy(
                    tile_v.at[i, :, pl.ds(0, 128)],
                    out_hbm.at[s0 + s_l, i, pl.ds(n_j * 8 * bg, n_j * 8)],
                    wsem,
                )
                for i in range(n_i)
            ]

        idx_desc(0).start()
        idx_desc(0).wait()
        remap(0)
        idx_desc(1).start()
        for d in gather_descs(0):
            d.start()

        def step(s_l, carry):
            rbuf = lax.rem(s_l, 2)

            @pl.when(s_l < s_per - 1)
            def _():
                idx_desc(s_l + 1).wait()
                remap(s_l + 1)
                for d in gather_descs(s_l + 1):
                    d.start()

            for d in gather_descs(s_l):
                d.wait()

            @pl.when(s_l < s_per - 2)
            def _():
                idx_desc(s_l + 2).start()

            @pl.when(s_l >= 1)
            def _():
                for d in write_descs(s_l - 1):
                    d.wait()

            def tr(t, carry2):
                for bi in range(4):
                    b = t * 4 + bi
                    jl = b >> 7
                    c = b & 127
                    cvec = jnp.broadcast_to(c, (16,))
                    midv = jnp.broadcast_to(jl * 8, (16,)) + QV
                    va = rows_v[rbuf, b, pl.ds(0, 16)]
                    vb = rows_v[rbuf, b, pl.ds(16, 16)]
                    plsc.store_scatter(tile_v, [I_LO, midv, cvec], va)
                    plsc.store_scatter(tile_v, [I_HI, midv, cvec], vb)
                return carry2

            lax.fori_loop(0, b_per // 4, tr, None)

            for d in write_descs(s_l):
                d.start()
            return carry

        lax.fori_loop(0, s_per, step, None)
        for d in write_descs(s_per - 1):
            d.wait()

    return gather_kernel(features, table_lin)


def _tc_tile_init(init_state, batch):
    L, _, H = init_state.shape
    blk = 512
    grid = (batch // blk,)

    def tile_kernel(init_ref, out_ref):
        out_ref[...] = jnp.broadcast_to(init_ref[...], out_ref.shape)

    return pl.pallas_call(
        tile_kernel,
        grid=grid,
        in_specs=[pl.BlockSpec((L, 1, H), lambda i: (0, 0, 0))],
        out_specs=pl.BlockSpec((L, blk, H), lambda i: (0, i, 0)),
        out_shape=jax.ShapeDtypeStruct((L, batch, H), init_state.dtype),
    )(init_state)


def kernel(features, lengths, table, init_state):
    del lengths
    V, D = table.shape
    t_pack = _tc_pack_table(jnp.transpose(table))
    table_lin = jnp.reshape(t_pack, (t_pack.shape[0] * (128 // D), D))
    B = features.shape[1]
    S = features.shape[0]
    out4 = _sc_gather(features, table_lin)
    emb5 = jnp.reshape(out4, (S, D // 8, B // 128, 8, 128))
    emb = jnp.transpose(emb5, (2, 4, 0, 1, 3)).reshape(B, S, D)
    init = _tc_tile_init(init_state, features.shape[1])
    return (emb, init)


# speedup vs baseline: 3.0809x; 1.0031x over previous
import functools

import jax
import jax.numpy as jnp
from jax import lax
from jax.experimental import pallas as pl
from jax.experimental.pallas import tpu as pltpu
from jax.experimental.pallas import tpu_sc as plsc

_NC = 2
_NS = 16
_NW = _NC * _NS
_BG = 4
_SG = 8
_IDX_CHUNK = 512
_PBLK = 8192
_VBLK = 512
_VQ = _VBLK // 4
_VQ_LOG2 = _VQ.bit_length() - 1


def _tc_pack_table(table_t):
    D, V = table_t.shape
    g = 128 // D
    v_blk = _PBLK
    rows_blk = v_blk // g
    grid = (pl.cdiv(V, v_blk),)
    n_rows = grid[0] * rows_blk

    def pack_kernel(in_ref, out_ref):
        a = in_ref[...]
        for w in range(v_blk // (4 * 128)):
            sub = a[:, w * 512:(w + 1) * 512]
            c = jnp.concatenate(
                [sub[:, m * 128:(m + 1) * 128] for m in range(4)], axis=0
            )
            out_ref[w * 128:(w + 1) * 128, :] = c.T

    return pl.pallas_call(
        pack_kernel,
        grid=grid,
        in_specs=[pl.BlockSpec((D, v_blk), lambda i: (0, i))],
        out_specs=pl.BlockSpec((rows_blk, 128), lambda i: (i, 0)),
        out_shape=jax.ShapeDtypeStruct((n_rows, 128), table_t.dtype),
    )(table_t)


def _sc_gather(features, table_lin):
    S, B = features.shape
    V, D = table_lin.shape
    s_per = S // _SG
    b_per = B // _BG
    n_ch = b_per // _IDX_CHUNK
    n_i = D // 8
    n_j = b_per // 128
    TP = 129

    mesh = plsc.VectorSubcoreMesh(core_axis_name="c", subcore_axis_name="s")

    @functools.partial(
        pl.kernel,
        out_type=jax.ShapeDtypeStruct((S, n_i, (B // 128) * 8, 128), jnp.float32),
        mesh=mesh,
        scratch_types=[
            pltpu.VMEM((2, b_per), jnp.int32),
            pltpu.VMEM((2, b_per, D), jnp.float32),
            pltpu.VMEM((n_i, n_j * 8, TP), jnp.float32),
            pltpu.SemaphoreType.DMA,
            pltpu.SemaphoreType.DMA((2,)),
            pltpu.SemaphoreType.DMA,
        ],
        compiler_params=pltpu.CompilerParams(
            use_tc_tiling_on_sc=False, needs_layout_passes=False
        ),
    )
    def gather_kernel(feat_hbm, table_hbm, out_hbm, idx_v, rows_v, tile_v,
                      isem, gsem, wsem):
        wid = lax.axis_index("c") * _NS + lax.axis_index("s")
        bg = wid // _SG
        sg = wid % _SG
        s0 = sg * s_per
        b0 = bg * b_per
        iota16 = lax.broadcasted_iota(jnp.int32, (16,), 0)
        I_LO = iota16 >> 3
        I_HI = I_LO + 2
        QV = iota16 & 7

        def idx_desc(s_l):
            return pltpu.make_async_copy(
                feat_hbm.at[s0 + s_l, pl.ds(b0, b_per)],
                idx_v.at[lax.rem(s_l, 2)],
                isem,
            )

        def remap(s_l):
            slot = lax.rem(s_l, 2)

            def body(i, carry):
                v = idx_v[slot, pl.ds(i * 16, 16)]
                k = (v & -_VBLK) | ((v & (_VQ - 1)) << 2) | ((v >> _VQ_LOG2) & 3)
                idx_v[slot, pl.ds(i * 16, 16)] = k
                return carry

            lax.fori_loop(0, b_per // 16, body, None)

        def gather_descs(s_l):
            slot = lax.rem(s_l, 2)
            return [
                pltpu.make_async_copy(
                    table_hbm.at[idx_v.at[slot, pl.ds(j * _IDX_CHUNK, _IDX_CHUNK)]],
                    rows_v.at[slot, pl.ds(j * _IDX_CHUNK, _IDX_CHUNK)],
                    gsem.at[slot],
                )
                for j in range(n_ch)
            ]

        def write_descs(s_l):
            return [
                pltpu.make_async_copy(
                    tile_v.at[i, :, pl.ds(0, 128)],
                    out_hbm.at[s0 + s_l, i, pl.ds(n_j * 8 * bg, n_j * 8)],
                    wsem,
                )
                for i in range(n_i)
            ]

        idx_desc(0).start()
        idx_desc(0).wait()
        remap(0)
        idx_desc(1).start()
        for d in gather_descs(0):
            d.start()

        def step(s_l, carry):
            rbuf = lax.rem(s_l, 2)

            @pl.when(s_l < s_per - 1)
            def _():
                idx_desc(s_l + 1).wait()
                remap(s_l + 1)
                for d in gather_descs(s_l + 1):
                    d.start()

            for d in gather_descs(s_l):
                d.wait()

            @pl.when(s_l < s_per - 2)
            def _():
                idx_desc(s_l + 2).start()

            @pl.when(s_l >= 1)
            def _():
                for d in write_descs(s_l - 1):
                    d.wait()

            def tr(t, carry2):
                for bi in range(8):
                    b = t * 8 + bi
                    jl = b >> 7
                    c = b & 127
                    cvec = jnp.broadcast_to(c, (16,))
                    midv = jnp.broadcast_to(jl * 8, (16,)) + QV
                    va = rows_v[rbuf, b, pl.ds(0, 16)]
                    vb = rows_v[rbuf, b, pl.ds(16, 16)]
                    plsc.store_scatter(tile_v, [I_LO, midv, cvec], va)
                    plsc.store_scatter(tile_v, [I_HI, midv, cvec], vb)
                return carry2

            lax.fori_loop(0, b_per // 8, tr, None)

            for d in write_descs(s_l):
                d.start()
            return carry

        lax.fori_loop(0, s_per, step, None)
        for d in write_descs(s_per - 1):
            d.wait()

    return gather_kernel(features, table_lin)


def _tc_tile_init(init_state, batch):
    L, _, H = init_state.shape
    blk = 512
    grid = (batch // blk,)

    def tile_kernel(init_ref, out_ref):
        out_ref[...] = jnp.broadcast_to(init_ref[...], out_ref.shape)

    return pl.pallas_call(
        tile_kernel,
        grid=grid,
        in_specs=[pl.BlockSpec((L, 1, H), lambda i: (0, 0, 0))],
        out_specs=pl.BlockSpec((L, blk, H), lambda i: (0, i, 0)),
        out_shape=jax.ShapeDtypeStruct((L, batch, H), init_state.dtype),
    )(init_state)


def kernel(features, lengths, table, init_state):
    del lengths
    V, D = table.shape
    t_pack = _tc_pack_table(jnp.transpose(table))
    table_lin = jnp.reshape(t_pack, (t_pack.shape[0] * (128 // D), D))
    B = features.shape[1]
    S = features.shape[0]
    out4 = _sc_gather(features, table_lin)
    emb5 = jnp.reshape(out4, (S, D // 8, B // 128, 8, 128))
    emb = jnp.transpose(emb5, (2, 4, 0, 1, 3)).reshape(B, S, D)
    init = _tc_tile_init(init_state, features.shape[1])
    return (emb, init)
